# baseline (speedup 1.0000x reference)
import functools

import jax
import jax.numpy as jnp
from jax import lax
from jax.experimental import pallas as pl
from jax.experimental.pallas import tpu as pltpu
from jax.experimental.pallas import tpu_sc as plsc

N = 10000
E = 320000
NC, NS = 2, 16
EPS = E // NS
C = 40
NCH = EPS // C
NPAD = 10240
RPT = NPAD // NS

BE = 4000
NEB = E // BE
BN_ = 400
NNB = N // BN_


def _edge1(edge_attr, We1, be1):
  def body(a_ref, w_ref, b_ref, o_ref):
    o_ref[...] = jnp.dot(a_ref[...], w_ref[...],
                         preferred_element_type=jnp.float32) + b_ref[...]

  return pl.pallas_call(
      body,
      grid=(NEB,),
      in_specs=[
          pl.BlockSpec((BE, 16), lambda j: (j, 0)),
          pl.BlockSpec((16, 128), lambda j: (0, 0)),
          pl.BlockSpec((1, 128), lambda j: (0, 0)),
      ],
      out_specs=pl.BlockSpec((BE, 128), lambda j: (j, 0)),
      out_shape=jax.ShapeDtypeStruct((E, 128), jnp.float32),
  )(edge_attr, We1, be1.reshape(1, 128))


def _edge2(edge_attr, We2, be2):
  def body(a_ref, w_ref, b_ref, o_ref):
    o_ref[...] = jnp.dot(a_ref[...], w_ref[...],
                         preferred_element_type=jnp.float32) + b_ref[...]

  return pl.pallas_call(
      body,
      grid=(NEB, NC),
      in_specs=[
          pl.BlockSpec((BE, 16), lambda j, c: (j, 0)),
          pl.BlockSpec((16, 128), lambda j, c: (0, c)),
          pl.BlockSpec((1, 128), lambda j, c: (0, c)),
      ],
      out_specs=pl.BlockSpec((BE, 128), lambda j, c: (c * NEB + j, 0)),
      out_shape=jax.ShapeDtypeStruct((2 * E, 128), jnp.float32),
  )(edge_attr, We2, be2.reshape(1, 256))


def _sc_aggregate(xsrc, eee, srcidx, dstidx, nch, cstride, sstride):
  KF = 8
  SB = 25
  NSB = nch // SB

  @functools.partial(
      pl.kernel,
      out_type=jax.ShapeDtypeStruct((2 * NPAD, 128), jnp.float32),
      mesh=plsc.VectorSubcoreMesh(
          core_axis_name="c", subcore_axis_name="s", num_cores=NC, num_subcores=NS),
      scratch_types=[
          pltpu.VMEM((SB, C), jnp.int32),
          pltpu.VMEM((SB, C), jnp.int32),
          pltpu.VMEM((C, 128), jnp.float32),
          pltpu.VMEM((C, 128), jnp.float32),
          pltpu.VMEM((C, 128), jnp.float32),
          pltpu.VMEM((C, 128), jnp.float32),
          pltpu.VMEM((C, 128), jnp.float32),
          pltpu.VMEM((C, 128), jnp.float32),
          pltpu.VMEM_SHARED((NPAD, 128), jnp.float32),
          pltpu.SemaphoreType.DMA,
          pltpu.SemaphoreType.DMA,
          pltpu.SemaphoreType.DMA,
          pltpu.SemaphoreType.DMA,
      ],
  )
  def k(x_hbm, e_hbm, src_hbm, dst_hbm, out_hbm,
        src_v, dst_v, x_a, e_a, m_a, x_b, e_b, m_b, agg_s, ls_a, ls_b, ss_a, ss_b):
    cid = lax.axis_index("c")
    sid = lax.axis_index("s")

    def zrow(r, _):
      for kk in range(KF):
        x_a[r, pl.ds(kk * 16, 16)] = jnp.zeros((16,), jnp.float32)
      return 0
    lax.fori_loop(0, C, zrow, 0)
    zbase = sid * RPT
    for t in range(RPT // C):
      pltpu.sync_copy(x_a, agg_s.at[pl.ds(zbase + t * C, C)])
    plsc.subcore_barrier()

    ebase0 = cid * cstride + sid * sstride

    def superstep(b, _):
      flat = (cid * NS + sid) * NSB + b
      pltpu.sync_copy(src_hbm.at[flat], src_v)
      pltpu.sync_copy(dst_hbm.at[flat], dst_v)
      eb = ebase0 + b * SB * C

      def issue_loads(i, xbuf, ebuf, lsem):
        pltpu.async_copy(x_hbm.at[src_v.at[i]], xbuf, lsem)
        pltpu.async_copy(e_hbm.at[pl.ds(eb + i * C, C)], ebuf, lsem)

      def wait_loads(xbuf, ebuf, lsem):
        pltpu.make_async_copy(x_hbm.at[src_v.at[0]], xbuf, lsem).wait()
        pltpu.make_async_copy(e_hbm.at[pl.ds(0, C)], ebuf, lsem).wait()

      def compute(xbuf, ebuf, mbuf):
        def crow(r, _):
          for kk in range(KF):
            sl = pl.ds(kk * 16, 16)
            mbuf[r, sl] = jnp.maximum(xbuf[r, sl] + ebuf[r, sl], 0.0)
          return 0
        lax.fori_loop(0, C, crow, 0)

      def issue_scatter(i, mbuf, ssem):
        pltpu.async_copy(mbuf, agg_s.at[dst_v.at[i]], ssem, add=True)

      def wait_scatter(mbuf, ssem):
        pltpu.make_async_copy(mbuf, agg_s.at[dst_v.at[0]], ssem).wait()

      issue_loads(0, x_a, e_a, ls_a)
      wait_loads(x_a, e_a, ls_a)
      issue_loads(1, x_b, e_b, ls_b)
      compute(x_a, e_a, m_a)
      issue_scatter(0, m_a, ss_a)
      wait_loads(x_b, e_b, ls_b)
      issue_loads(2, x_a, e_a, ls_a)
      compute(x_b, e_b, m_b)
      issue_scatter(1, m_b, ss_b)

      def pair(kk, _):
        j0 = 2 * kk
        wait_loads(x_a, e_a, ls_a)
        issue_loads(j0 + 1, x_b, e_b, ls_b)
        wait_scatter(m_a, ss_a)
        compute(x_a, e_a, m_a)
        issue_scatter(j0, m_a, ss_a)
        wait_loads(x_b, e_b, ls_b)
        issue_loads(j0 + 2, x_a, e_a, ls_a)
        wait_scatter(m_b, ss_b)
        compute(x_b, e_b, m_b)
        issue_scatter(j0 + 1, m_b, ss_b)
        return 0
      lax.fori_loop(1, 12, pair, 0)

      wait_loads(x_a, e_a, ls_a)
      wait_scatter(m_a, ss_a)
      compute(x_a, e_a, m_a)
      issue_scatter(24, m_a, ss_a)
      wait_scatter(m_b, ss_b)
      wait_scatter(m_a, ss_a)
      return 0
    lax.fori_loop(0, NSB, superstep, 0)
    plsc.subcore_barrier()

    wbase = cid * NPAD + sid * RPT
    for t in range(RPT // C):
      pltpu.sync_copy(agg_s.at[pl.ds(zbase + t * C, C)], x_a)
      pltpu.sync_copy(x_a, out_hbm.at[pl.ds(wbase + t * C, C)])

  return k(xsrc, eee, srcidx, dstidx)


def _mlp1(x, alo, ahi, W11, b11, W12, b12, sc1, sh1):
  def body(x_ref, alo_ref, ahi_ref, w1_ref, b1_ref, w2_ref, b2_ref,
           sc_ref, sh_ref, o_ref):
    h = x_ref[...] + alo_ref[...] + ahi_ref[...]
    t = jax.nn.relu(jnp.dot(h, w1_ref[...], preferred_element_type=jnp.float32) + b1_ref[...])
    u = jnp.dot(t, w2_ref[...], preferred_element_type=jnp.float32) + b2_ref[...]
    o_ref[...] = sc_ref[...] * jax.nn.relu(u) + sh_ref[...]

  return pl.pallas_call(
      body,
      grid=(NNB, NC),
      in_specs=[
          pl.BlockSpec((BN_, 128), lambda j, c: (j, 0)),
          pl.BlockSpec((BN_, 128), lambda j, c: (j, 0)),
          pl.BlockSpec((BN_, 128), lambda j, c: (j, 0)),
          pl.BlockSpec((128, 256), lambda j, c: (0, 0)),
          pl.BlockSpec((1, 256), lambda j, c: (0, 0)),
          pl.BlockSpec((256, 128), lambda j, c: (0, c)),
          pl.BlockSpec((1, 128), lambda j, c: (0, c)),
          pl.BlockSpec((1, 128), lambda j, c: (0, c)),
          pl.BlockSpec((1, 128), lambda j, c: (0, c)),
      ],
      out_specs=pl.BlockSpec((BN_, 128), lambda j, c: (c * NNB + j, 0)),
      out_shape=jax.ShapeDtypeStruct((2 * N, 128), jnp.float32),
  )(x, alo, ahi, W11, b11.reshape(1, 256), W12, b12.reshape(1, 256),
    sc1.reshape(1, 256), sh1.reshape(1, 256))


def _mlp2_final(h1, alo, ahi, W21, b21, W22, b22, sc2, sh2, Wl, bl):
  def body(hlo_ref, hhi_ref, alo_ref, ahi_ref, w1_ref, b1_ref, w2_ref, b2_ref,
           sc_ref, sh_ref, wl_ref, bl_ref, o_ref):
    h = jnp.concatenate([hlo_ref[...], hhi_ref[...]], axis=1)
    a = jnp.concatenate([alo_ref[...], ahi_ref[...]], axis=1)
    z = h + a
    t = jax.nn.relu(jnp.dot(z, w1_ref[...], preferred_element_type=jnp.float32) + b1_ref[...])
    u = jnp.dot(t, w2_ref[...], preferred_element_type=jnp.float32) + b2_ref[...]
    v = sc_ref[...] * jax.nn.relu(u) + sh_ref[...]
    o_ref[...] = jnp.dot(v, wl_ref[...], preferred_element_type=jnp.float32) + bl_ref[...]

  return pl.pallas_call(
      body,
      grid=(NNB,),
      in_specs=[
          pl.BlockSpec((BN_, 128), lambda j: (j, 0)),
          pl.BlockSpec((BN_, 128), lambda j: (NNB + j, 0)),
          pl.BlockSpec((BN_, 128), lambda j: (j, 0)),
          pl.BlockSpec((BN_, 128), lambda j: (j, 0)),
          pl.BlockSpec((256, 256), lambda j: (0, 0)),
          pl.BlockSpec((1, 256), lambda j: (0, 0)),
          pl.BlockSpec((256, 256), lambda j: (0, 0)),
          pl.BlockSpec((1, 256), lambda j: (0, 0)),
          pl.BlockSpec((1, 256), lambda j: (0, 0)),
          pl.BlockSpec((1, 256), lambda j: (0, 0)),
          pl.BlockSpec((256, 128), lambda j: (0, 0)),
          pl.BlockSpec((1, 128), lambda j: (0, 0)),
      ],
      out_specs=pl.BlockSpec((BN_, 128), lambda j: (j, 0)),
      out_shape=jax.ShapeDtypeStruct((N, 128), jnp.float32),
  )(h1, h1, alo, ahi, W21, b21.reshape(1, 256), W22, b22.reshape(1, 256),
    sc2.reshape(1, 256), sh2.reshape(1, 256), Wl, bl.reshape(1, 128))


def kernel(x, edge_index, edge_attr, We1, be1, W11, b11, W12, b12,
           We2, be2, W21, b21, W22, b22, g1, bb1, g2, bb2, Wl, bl):
  src = edge_index[0]
  dst = edge_index[1]

  nch1 = E // (NC * NS) // C
  src1 = src.reshape(-1, 25, C)
  dst1 = dst.reshape(-1, 25, C)

  src_r = src.reshape(NS, NCH, C)
  src2 = jnp.stack([src_r, src_r + N]).reshape(-1, 25, C)
  dst_r = dst.reshape(NS, NCH, C)
  dst2 = jnp.stack([dst_r, dst_r]).reshape(-1, 25, C)

  inv = 1.0 / jnp.sqrt(jnp.float32(1.0 + 1e-5))
  sc1, sh1 = g1 * inv, bb1
  sc2, sh2 = g2 * inv, bb2

  e1 = _edge1(edge_attr, We1, be1)
  e2 = _edge2(edge_attr, We2, be2)

  agg1 = _sc_aggregate(x, e1, src1, dst1, nch1, E // NC, E // (NC * NS))
  h1 = _mlp1(x, agg1[:N], agg1[NPAD:NPAD + N],
             W11, b11, W12, b12, sc1, sh1)

  agg2 = _sc_aggregate(h1, e2, src2, dst2, NCH, E, EPS)
  return _mlp2_final(h1, agg2[:N], agg2[NPAD:NPAD + N],
                     W21, b21, W22, b22, sc2, sh2, Wl, bl)


# speedup vs baseline: 1.2536x; 1.2536x over previous
import functools

import jax
import jax.numpy as jnp
from jax import lax
from jax.experimental import pallas as pl
from jax.experimental.pallas import tpu as pltpu
from jax.experimental.pallas import tpu_sc as plsc

N = 10000
E = 320000
NC, NS = 2, 16
EPS = E // NS
C = 80
NCH = EPS // C
NPAD = 10240
RPT = NPAD // NS

BE = 4000
NEB = E // BE
BN_ = 400
NNB = N // BN_


def _edge1(edge_attr, We1, be1):
  def body(a_ref, w_ref, b_ref, o_ref):
    o_ref[...] = jnp.dot(a_ref[...], w_ref[...],
                         preferred_element_type=jnp.float32) + b_ref[...]

  return pl.pallas_call(
      body,
      grid=(NEB,),
      in_specs=[
          pl.BlockSpec((BE, 16), lambda j: (j, 0)),
          pl.BlockSpec((16, 128), lambda j: (0, 0)),
          pl.BlockSpec((1, 128), lambda j: (0, 0)),
      ],
      out_specs=pl.BlockSpec((BE, 128), lambda j: (j, 0)),
      out_shape=jax.ShapeDtypeStruct((E, 128), jnp.float32),
  )(edge_attr, We1, be1.reshape(1, 128))


def _edge2(edge_attr, We2, be2):
  def body(a_ref, w_ref, b_ref, o_ref):
    o_ref[...] = jnp.dot(a_ref[...], w_ref[...],
                         preferred_element_type=jnp.float32) + b_ref[...]

  return pl.pallas_call(
      body,
      grid=(NEB, NC),
      in_specs=[
          pl.BlockSpec((BE, 16), lambda j, c: (j, 0)),
          pl.BlockSpec((16, 128), lambda j, c: (0, c)),
          pl.BlockSpec((1, 128), lambda j, c: (0, c)),
      ],
      out_specs=pl.BlockSpec((BE, 128), lambda j, c: (c * NEB + j, 0)),
      out_shape=jax.ShapeDtypeStruct((2 * E, 128), jnp.float32),
  )(edge_attr, We2, be2.reshape(1, 256))


def _sc_aggregate(xsrc, eee, srcidx, dstidx, nch, cstride, sstride):
  KF = 8
  SB = 25
  NSB = nch // SB

  @functools.partial(
      pl.kernel,
      out_type=jax.ShapeDtypeStruct((2 * NPAD, 128), jnp.float32),
      mesh=plsc.VectorSubcoreMesh(
          core_axis_name="c", subcore_axis_name="s", num_cores=NC, num_subcores=NS),
      scratch_types=[
          pltpu.VMEM((SB, C), jnp.int32),
          pltpu.VMEM((SB, C), jnp.int32),
          pltpu.VMEM((C, 128), jnp.float32),
          pltpu.VMEM((C, 128), jnp.float32),
          pltpu.VMEM((C, 128), jnp.float32),
          pltpu.VMEM((C, 128), jnp.float32),
          pltpu.VMEM_SHARED((NPAD, 128), jnp.float32),
          pltpu.SemaphoreType.DMA,
          pltpu.SemaphoreType.DMA,
          pltpu.SemaphoreType.DMA,
          pltpu.SemaphoreType.DMA,
      ],
  )
  def k(x_hbm, e_hbm, src_hbm, dst_hbm, out_hbm,
        src_v, dst_v, x_a, e_a, x_b, e_b, agg_s, ls_a, ls_b, ss_a, ss_b):
    cid = lax.axis_index("c")
    sid = lax.axis_index("s")

    def zrow(r, _):
      for kk in range(KF):
        x_a[r, pl.ds(kk * 16, 16)] = jnp.zeros((16,), jnp.float32)
      return 0
    lax.fori_loop(0, C, zrow, 0)
    zbase = sid * RPT
    for t in range(8):
      pltpu.sync_copy(x_a, agg_s.at[pl.ds(zbase + t * C, C)])
    plsc.subcore_barrier()

    ebase0 = cid * cstride + sid * sstride

    def superstep(b, _):
      flat = (cid * NS + sid) * NSB + b
      pltpu.sync_copy(src_hbm.at[flat], src_v)
      pltpu.sync_copy(dst_hbm.at[flat], dst_v)
      eb = ebase0 + b * SB * C

      def issue_loads(i, xbuf, ebuf, lsem):
        pltpu.async_copy(x_hbm.at[src_v.at[i]], xbuf, lsem)
        pltpu.async_copy(e_hbm.at[pl.ds(eb + i * C, C)], ebuf, lsem)

      def wait_loads(xbuf, ebuf, lsem):
        pltpu.make_async_copy(x_hbm.at[src_v.at[0]], xbuf, lsem).wait()
        pltpu.make_async_copy(e_hbm.at[pl.ds(0, C)], ebuf, lsem).wait()

      def compute(xbuf, ebuf):
        def crow(r4, _):
          r = r4 * 4
          for dr in range(4):
            for kk in range(KF):
              sl = pl.ds(kk * 16, 16)
              ebuf[r + dr, sl] = jnp.maximum(
                  xbuf[r + dr, sl] + ebuf[r + dr, sl], 0.0)
          return 0
        lax.fori_loop(0, C // 4, crow, 0)

      def issue_scatter(i, ebuf, ssem):
        pltpu.async_copy(ebuf, agg_s.at[dst_v.at[i]], ssem, add=True)

      def wait_scatter(ebuf, ssem):
        pltpu.make_async_copy(ebuf, agg_s.at[dst_v.at[0]], ssem).wait()

      issue_loads(0, x_a, e_a, ls_a)
      wait_loads(x_a, e_a, ls_a)
      issue_loads(1, x_b, e_b, ls_b)
      compute(x_a, e_a)
      issue_scatter(0, e_a, ss_a)
      wait_loads(x_b, e_b, ls_b)
      issue_loads(2, x_a, e_a, ls_a)
      compute(x_b, e_b)
      issue_scatter(1, e_b, ss_b)

      def pair(kk, _):
        j0 = 2 * kk
        wait_loads(x_a, e_a, ls_a)
        issue_loads(j0 + 1, x_b, e_b, ls_b)
        wait_scatter(e_a, ss_a)
        compute(x_a, e_a)
        issue_scatter(j0, e_a, ss_a)
        wait_loads(x_b, e_b, ls_b)
        issue_loads(j0 + 2, x_a, e_a, ls_a)
        wait_scatter(e_b, ss_b)
        compute(x_b, e_b)
        issue_scatter(j0 + 1, e_b, ss_b)
        return 0
      lax.fori_loop(1, 12, pair, 0)

      wait_loads(x_a, e_a, ls_a)
      wait_scatter(e_a, ss_a)
      compute(x_a, e_a)
      issue_scatter(24, e_a, ss_a)
      wait_scatter(e_b, ss_b)
      wait_scatter(e_a, ss_a)
      return 0
    lax.fori_loop(0, NSB, superstep, 0)
    plsc.subcore_barrier()

    wbase = cid * NPAD + sid * RPT
    for t in range(8):
      pltpu.sync_copy(agg_s.at[pl.ds(zbase + t * C, C)], x_a)
      pltpu.sync_copy(x_a, out_hbm.at[pl.ds(wbase + t * C, C)])

  return k(xsrc, eee, srcidx, dstidx)


def _mlp1(x, alo, ahi, W11, b11, W12, b12, sc1, sh1):
  def body(x_ref, alo_ref, ahi_ref, w1_ref, b1_ref, w2_ref, b2_ref,
           sc_ref, sh_ref, o_ref):
    h = x_ref[...] + alo_ref[...] + ahi_ref[...]
    t = jax.nn.relu(jnp.dot(h, w1_ref[...], preferred_element_type=jnp.float32) + b1_ref[...])
    u = jnp.dot(t, w2_ref[...], preferred_element_type=jnp.float32) + b2_ref[...]
    o_ref[...] = sc_ref[...] * jax.nn.relu(u) + sh_ref[...]

  return pl.pallas_call(
      body,
      grid=(NNB, NC),
      in_specs=[
          pl.BlockSpec((BN_, 128), lambda j, c: (j, 0)),
          pl.BlockSpec((BN_, 128), lambda j, c: (j, 0)),
          pl.BlockSpec((BN_, 128), lambda j, c: (j, 0)),
          pl.BlockSpec((128, 256), lambda j, c: (0, 0)),
          pl.BlockSpec((1, 256), lambda j, c: (0, 0)),
          pl.BlockSpec((256, 128), lambda j, c: (0, c)),
          pl.BlockSpec((1, 128), lambda j, c: (0, c)),
          pl.BlockSpec((1, 128), lambda j, c: (0, c)),
          pl.BlockSpec((1, 128), lambda j, c: (0, c)),
      ],
      out_specs=pl.BlockSpec((BN_, 128), lambda j, c: (c * NNB + j, 0)),
      out_shape=jax.ShapeDtypeStruct((2 * N, 128), jnp.float32),
  )(x, alo, ahi, W11, b11.reshape(1, 256), W12, b12.reshape(1, 256),
    sc1.reshape(1, 256), sh1.reshape(1, 256))


def _mlp2_final(h1, alo, ahi, W21, b21, W22, b22, sc2, sh2, Wl, bl):
  def body(hlo_ref, hhi_ref, alo_ref, ahi_ref, w1_ref, b1_ref, w2_ref, b2_ref,
           sc_ref, sh_ref, wl_ref, bl_ref, o_ref):
    h = jnp.concatenate([hlo_ref[...], hhi_ref[...]], axis=1)
    a = jnp.concatenate([alo_ref[...], ahi_ref[...]], axis=1)
    z = h + a
    t = jax.nn.relu(jnp.dot(z, w1_ref[...], preferred_element_type=jnp.float32) + b1_ref[...])
    u = jnp.dot(t, w2_ref[...], preferred_element_type=jnp.float32) + b2_ref[...]
    v = sc_ref[...] * jax.nn.relu(u) + sh_ref[...]
    o_ref[...] = jnp.dot(v, wl_ref[...], preferred_element_type=jnp.float32) + bl_ref[...]

  return pl.pallas_call(
      body,
      grid=(NNB,),
      in_specs=[
          pl.BlockSpec((BN_, 128), lambda j: (j, 0)),
          pl.BlockSpec((BN_, 128), lambda j: (NNB + j, 0)),
          pl.BlockSpec((BN_, 128), lambda j: (j, 0)),
          pl.BlockSpec((BN_, 128), lambda j: (j, 0)),
          pl.BlockSpec((256, 256), lambda j: (0, 0)),
          pl.BlockSpec((1, 256), lambda j: (0, 0)),
          pl.BlockSpec((256, 256), lambda j: (0, 0)),
          pl.BlockSpec((1, 256), lambda j: (0, 0)),
          pl.BlockSpec((1, 256), lambda j: (0, 0)),
          pl.BlockSpec((1, 256), lambda j: (0, 0)),
          pl.BlockSpec((256, 128), lambda j: (0, 0)),
          pl.BlockSpec((1, 128), lambda j: (0, 0)),
      ],
      out_specs=pl.BlockSpec((BN_, 128), lambda j: (j, 0)),
      out_shape=jax.ShapeDtypeStruct((N, 128), jnp.float32),
  )(h1, h1, alo, ahi, W21, b21.reshape(1, 256), W22, b22.reshape(1, 256),
    sc2.reshape(1, 256), sh2.reshape(1, 256), Wl, bl.reshape(1, 128))


def kernel(x, edge_index, edge_attr, We1, be1, W11, b11, W12, b12,
           We2, be2, W21, b21, W22, b22, g1, bb1, g2, bb2, Wl, bl):
  src = edge_index[0]
  dst = edge_index[1]

  nch1 = E // (NC * NS) // C
  src1 = src.reshape(-1, 25, C)
  dst1 = dst.reshape(-1, 25, C)

  src_r = src.reshape(NS, NCH, C)
  src2 = jnp.stack([src_r, src_r + N]).reshape(-1, 25, C)
  dst_r = dst.reshape(NS, NCH, C)
  dst2 = jnp.stack([dst_r, dst_r]).reshape(-1, 25, C)

  inv = 1.0 / jnp.sqrt(jnp.float32(1.0 + 1e-5))
  sc1, sh1 = g1 * inv, bb1
  sc2, sh2 = g2 * inv, bb2

  e1 = _edge1(edge_attr, We1, be1)
  e2 = _edge2(edge_attr, We2, be2)

  agg1 = _sc_aggregate(x, e1, src1, dst1, nch1, E // NC, E // (NC * NS))
  h1 = _mlp1(x, agg1[:N], agg1[NPAD:NPAD + N],
             W11, b11, W12, b12, sc1, sh1)

  agg2 = _sc_aggregate(h1, e2, src2, dst2, NCH, E, EPS)
  return _mlp2_final(h1, agg2[:N], agg2[NPAD:NPAD + N],
                     W21, b21, W22, b22, sc2, sh2, Wl, bl)


# speedup vs baseline: 1.2923x; 1.0308x over previous
import functools

import jax
import jax.numpy as jnp
from jax import lax
from jax.experimental import pallas as pl
from jax.experimental.pallas import tpu as pltpu
from jax.experimental.pallas import tpu_sc as plsc

N = 10000
E = 320000
NC, NS = 2, 16
EPS = E // NS
C = 80
NCH = EPS // C
NPAD = 10240
RPT = NPAD // NS

BE = 8000
NEB = E // BE
BN_ = 2000
NNB = N // BN_


def _edge1(edge_attr, We1, be1):
  def body(a_ref, w_ref, b_ref, o_ref):
    o_ref[...] = jnp.dot(a_ref[...], w_ref[...],
                         preferred_element_type=jnp.float32) + b_ref[...]

  return pl.pallas_call(
      body,
      grid=(NEB,),
      in_specs=[
          pl.BlockSpec((BE, 16), lambda j: (j, 0)),
          pl.BlockSpec((16, 128), lambda j: (0, 0)),
          pl.BlockSpec((1, 128), lambda j: (0, 0)),
      ],
      out_specs=pl.BlockSpec((BE, 128), lambda j: (j, 0)),
      out_shape=jax.ShapeDtypeStruct((E, 128), jnp.float32),
  )(edge_attr, We1, be1.reshape(1, 128))


def _edge2(edge_attr, We2, be2):
  def body(a_ref, w_ref, b_ref, o_ref):
    o_ref[...] = jnp.dot(a_ref[...], w_ref[...],
                         preferred_element_type=jnp.float32) + b_ref[...]

  return pl.pallas_call(
      body,
      grid=(NEB, NC),
      in_specs=[
          pl.BlockSpec((BE, 16), lambda j, c: (j, 0)),
          pl.BlockSpec((16, 128), lambda j, c: (0, c)),
          pl.BlockSpec((1, 128), lambda j, c: (0, c)),
      ],
      out_specs=pl.BlockSpec((BE, 128), lambda j, c: (c * NEB + j, 0)),
      out_shape=jax.ShapeDtypeStruct((2 * E, 128), jnp.float32),
  )(edge_attr, We2, be2.reshape(1, 256))


def _sc_aggregate(xsrc, eee, srcidx, dstidx, nch, cstride, sstride):
  KF = 8
  SB = 25
  NSB = nch // SB

  @functools.partial(
      pl.kernel,
      out_type=jax.ShapeDtypeStruct((2 * NPAD, 128), jnp.float32),
      mesh=plsc.VectorSubcoreMesh(
          core_axis_name="c", subcore_axis_name="s", num_cores=NC, num_subcores=NS),
      scratch_types=[
          pltpu.VMEM((SB, C), jnp.int32),
          pltpu.VMEM((SB, C), jnp.int32),
          pltpu.VMEM((C, 128), jnp.float32),
          pltpu.VMEM((C, 128), jnp.float32),
          pltpu.VMEM((C, 128), jnp.float32),
          pltpu.VMEM((C, 128), jnp.float32),
          pltpu.VMEM_SHARED((NPAD, 128), jnp.float32),
          pltpu.SemaphoreType.DMA,
          pltpu.SemaphoreType.DMA,
          pltpu.SemaphoreType.DMA,
          pltpu.SemaphoreType.DMA,
      ],
  )
  def k(x_hbm, e_hbm, src_hbm, dst_hbm, out_hbm,
        src_v, dst_v, x_a, e_a, x_b, e_b, agg_s, ls_a, ls_b, ss_a, ss_b):
    cid = lax.axis_index("c")
    sid = lax.axis_index("s")

    def zrow(r, _):
      for kk in range(KF):
        x_a[r, pl.ds(kk * 16, 16)] = jnp.zeros((16,), jnp.float32)
      return 0
    lax.fori_loop(0, C, zrow, 0)
    zbase = sid * RPT
    for t in range(8):
      pltpu.sync_copy(x_a, agg_s.at[pl.ds(zbase + t * C, C)])
    plsc.subcore_barrier()

    ebase0 = cid * cstride + sid * sstride

    def superstep(b, _):
      flat = (cid * NS + sid) * NSB + b
      pltpu.sync_copy(src_hbm.at[flat], src_v)
      pltpu.sync_copy(dst_hbm.at[flat], dst_v)
      eb = ebase0 + b * SB * C

      def issue_loads(i, xbuf, ebuf, lsem):
        pltpu.async_copy(x_hbm.at[src_v.at[i]], xbuf, lsem)
        pltpu.async_copy(e_hbm.at[pl.ds(eb + i * C, C)], ebuf, lsem)

      def wait_loads(xbuf, ebuf, lsem):
        pltpu.make_async_copy(x_hbm.at[src_v.at[0]], xbuf, lsem).wait()
        pltpu.make_async_copy(e_hbm.at[pl.ds(0, C)], ebuf, lsem).wait()

      def compute(xbuf, ebuf):
        def crow(r4, _):
          r = r4 * 4
          for dr in range(4):
            for kk in range(KF):
              sl = pl.ds(kk * 16, 16)
              ebuf[r + dr, sl] = jnp.maximum(
                  xbuf[r + dr, sl] + ebuf[r + dr, sl], 0.0)
          return 0
        lax.fori_loop(0, C // 4, crow, 0)

      def issue_scatter(i, ebuf, ssem):
        pltpu.async_copy(ebuf, agg_s.at[dst_v.at[i]], ssem, add=True)

      def wait_scatter(ebuf, ssem):
        pltpu.make_async_copy(ebuf, agg_s.at[dst_v.at[0]], ssem).wait()

      issue_loads(0, x_a, e_a, ls_a)
      wait_loads(x_a, e_a, ls_a)
      issue_loads(1, x_b, e_b, ls_b)
      compute(x_a, e_a)
      issue_scatter(0, e_a, ss_a)
      wait_loads(x_b, e_b, ls_b)
      issue_loads(2, x_a, e_a, ls_a)
      compute(x_b, e_b)
      issue_scatter(1, e_b, ss_b)

      def pair(kk, _):
        j0 = 2 * kk
        wait_loads(x_a, e_a, ls_a)
        issue_loads(j0 + 1, x_b, e_b, ls_b)
        wait_scatter(e_a, ss_a)
        compute(x_a, e_a)
        issue_scatter(j0, e_a, ss_a)
        wait_loads(x_b, e_b, ls_b)
        issue_loads(j0 + 2, x_a, e_a, ls_a)
        wait_scatter(e_b, ss_b)
        compute(x_b, e_b)
        issue_scatter(j0 + 1, e_b, ss_b)
        return 0
      lax.fori_loop(1, 12, pair, 0)

      wait_loads(x_a, e_a, ls_a)
      wait_scatter(e_a, ss_a)
      compute(x_a, e_a)
      issue_scatter(24, e_a, ss_a)
      wait_scatter(e_b, ss_b)
      wait_scatter(e_a, ss_a)
      return 0
    lax.fori_loop(0, NSB, superstep, 0)
    plsc.subcore_barrier()

    wbase = cid * NPAD + sid * RPT
    for t in range(8):
      pltpu.sync_copy(agg_s.at[pl.ds(zbase + t * C, C)], x_a)
      pltpu.sync_copy(x_a, out_hbm.at[pl.ds(wbase + t * C, C)])

  return k(xsrc, eee, srcidx, dstidx)


def _mlp1(x, alo, ahi, W11, b11, W12, b12, sc1, sh1):
  def body(x_ref, alo_ref, ahi_ref, w1_ref, b1_ref, w2_ref, b2_ref,
           sc_ref, sh_ref, o_ref):
    h = x_ref[...] + alo_ref[...] + ahi_ref[...]
    t = jax.nn.relu(jnp.dot(h, w1_ref[...], preferred_element_type=jnp.float32) + b1_ref[...])
    u = jnp.dot(t, w2_ref[...], preferred_element_type=jnp.float32) + b2_ref[...]
    o_ref[...] = sc_ref[...] * jax.nn.relu(u) + sh_ref[...]

  return pl.pallas_call(
      body,
      grid=(NNB, NC),
      in_specs=[
          pl.BlockSpec((BN_, 128), lambda j, c: (j, 0)),
          pl.BlockSpec((BN_, 128), lambda j, c: (j, 0)),
          pl.BlockSpec((BN_, 128), lambda j, c: (j, 0)),
          pl.BlockSpec((128, 256), lambda j, c: (0, 0)),
          pl.BlockSpec((1, 256), lambda j, c: (0, 0)),
          pl.BlockSpec((256, 128), lambda j, c: (0, c)),
          pl.BlockSpec((1, 128), lambda j, c: (0, c)),
          pl.BlockSpec((1, 128), lambda j, c: (0, c)),
          pl.BlockSpec((1, 128), lambda j, c: (0, c)),
      ],
      out_specs=pl.BlockSpec((BN_, 128), lambda j, c: (c * NNB + j, 0)),
      out_shape=jax.ShapeDtypeStruct((2 * N, 128), jnp.float32),
  )(x, alo, ahi, W11, b11.reshape(1, 256), W12, b12.reshape(1, 256),
    sc1.reshape(1, 256), sh1.reshape(1, 256))


def _mlp2_final(h1, alo, ahi, W21, b21, W22, b22, sc2, sh2, Wl, bl):
  def body(hlo_ref, hhi_ref, alo_ref, ahi_ref, w1_ref, b1_ref, w2_ref, b2_ref,
           sc_ref, sh_ref, wl_ref, bl_ref, o_ref):
    h = jnp.concatenate([hlo_ref[...], hhi_ref[...]], axis=1)
    a = jnp.concatenate([alo_ref[...], ahi_ref[...]], axis=1)
    z = h + a
    t = jax.nn.relu(jnp.dot(z, w1_ref[...], preferred_element_type=jnp.float32) + b1_ref[...])
    u = jnp.dot(t, w2_ref[...], preferred_element_type=jnp.float32) + b2_ref[...]
    v = sc_ref[...] * jax.nn.relu(u) + sh_ref[...]
    o_ref[...] = jnp.dot(v, wl_ref[...], preferred_element_type=jnp.float32) + bl_ref[...]

  return pl.pallas_call(
      body,
      grid=(NNB,),
      in_specs=[
          pl.BlockSpec((BN_, 128), lambda j: (j, 0)),
          pl.BlockSpec((BN_, 128), lambda j: (NNB + j, 0)),
          pl.BlockSpec((BN_, 128), lambda j: (j, 0)),
          pl.BlockSpec((BN_, 128), lambda j: (j, 0)),
          pl.BlockSpec((256, 256), lambda j: (0, 0)),
          pl.BlockSpec((1, 256), lambda j: (0, 0)),
          pl.BlockSpec((256, 256), lambda j: (0, 0)),
          pl.BlockSpec((1, 256), lambda j: (0, 0)),
          pl.BlockSpec((1, 256), lambda j: (0, 0)),
          pl.BlockSpec((1, 256), lambda j: (0, 0)),
          pl.BlockSpec((256, 128), lambda j: (0, 0)),
          pl.BlockSpec((1, 128), lambda j: (0, 0)),
      ],
      out_specs=pl.BlockSpec((BN_, 128), lambda j: (j, 0)),
      out_shape=jax.ShapeDtypeStruct((N, 128), jnp.float32),
  )(h1, h1, alo, ahi, W21, b21.reshape(1, 256), W22, b22.reshape(1, 256),
    sc2.reshape(1, 256), sh2.reshape(1, 256), Wl, bl.reshape(1, 128))


def kernel(x, edge_index, edge_attr, We1, be1, W11, b11, W12, b12,
           We2, be2, W21, b21, W22, b22, g1, bb1, g2, bb2, Wl, bl):
  src = edge_index[0]
  dst = edge_index[1]

  nch1 = E // (NC * NS) // C
  src1 = src.reshape(-1, 25, C)
  dst1 = dst.reshape(-1, 25, C)

  src_r = src.reshape(NS, NCH, C)
  src2 = jnp.stack([src_r, src_r + N]).reshape(-1, 25, C)
  dst_r = dst.reshape(NS, NCH, C)
  dst2 = jnp.stack([dst_r, dst_r]).reshape(-1, 25, C)

  inv = 1.0 / jnp.sqrt(jnp.float32(1.0 + 1e-5))
  sc1, sh1 = g1 * inv, bb1
  sc2, sh2 = g2 * inv, bb2

  e1 = _edge1(edge_attr, We1, be1)
  e2 = _edge2(edge_attr, We2, be2)

  agg1 = _sc_aggregate(x, e1, src1, dst1, nch1, E // NC, E // (NC * NS))
  h1 = _mlp1(x, agg1[:N], agg1[NPAD:NPAD + N],
             W11, b11, W12, b12, sc1, sh1)

  agg2 = _sc_aggregate(h1, e2, src2, dst2, NCH, E, EPS)
  return _mlp2_final(h1, agg2[:N], agg2[NPAD:NPAD + N],
                     W21, b21, W22, b22, sc2, sh2, Wl, bl)


# speedup vs baseline: 1.3029x; 1.0082x over previous
import functools

import jax
import jax.numpy as jnp
from jax import lax
from jax.experimental import pallas as pl
from jax.experimental.pallas import tpu as pltpu
from jax.experimental.pallas import tpu_sc as plsc

N = 10000
E = 320000
NC, NS = 2, 16
EPS = E // NS
C = 80
NCH = EPS // C
NPAD = 10240
RPT = NPAD // NS

BE = 16000
NEB = E // BE
BN_ = 5000
NNB = N // BN_


def _edge1(edge_attr, We1, be1):
  def body(a_ref, w_ref, b_ref, o_ref):
    o_ref[...] = jnp.dot(a_ref[...], w_ref[...],
                         preferred_element_type=jnp.float32) + b_ref[...]

  return pl.pallas_call(
      body,
      grid=(NEB,),
      in_specs=[
          pl.BlockSpec((BE, 16), lambda j: (j, 0)),
          pl.BlockSpec((16, 128), lambda j: (0, 0)),
          pl.BlockSpec((1, 128), lambda j: (0, 0)),
      ],
      out_specs=pl.BlockSpec((BE, 128), lambda j: (j, 0)),
      out_shape=jax.ShapeDtypeStruct((E, 128), jnp.float32),
  )(edge_attr, We1, be1.reshape(1, 128))


def _edge2(edge_attr, We2, be2):
  def body(a_ref, w_ref, b_ref, o_ref):
    o_ref[...] = jnp.dot(a_ref[...], w_ref[...],
                         preferred_element_type=jnp.float32) + b_ref[...]

  return pl.pallas_call(
      body,
      grid=(NEB, NC),
      in_specs=[
          pl.BlockSpec((BE, 16), lambda j, c: (j, 0)),
          pl.BlockSpec((16, 128), lambda j, c: (0, c)),
          pl.BlockSpec((1, 128), lambda j, c: (0, c)),
      ],
      out_specs=pl.BlockSpec((BE, 128), lambda j, c: (c * NEB + j, 0)),
      out_shape=jax.ShapeDtypeStruct((2 * E, 128), jnp.float32),
  )(edge_attr, We2, be2.reshape(1, 256))


def _sc_aggregate(xsrc, eee, srcidx, dstidx, nch, cstride, sstride):
  KF = 8
  SB = 25
  NSB = nch // SB

  @functools.partial(
      pl.kernel,
      out_type=jax.ShapeDtypeStruct((2 * NPAD, 128), jnp.float32),
      mesh=plsc.VectorSubcoreMesh(
          core_axis_name="c", subcore_axis_name="s", num_cores=NC, num_subcores=NS),
      scratch_types=[
          pltpu.VMEM((SB, C), jnp.int32),
          pltpu.VMEM((SB, C), jnp.int32),
          pltpu.VMEM((C, 128), jnp.float32),
          pltpu.VMEM((C, 128), jnp.float32),
          pltpu.VMEM((C, 128), jnp.float32),
          pltpu.VMEM((C, 128), jnp.float32),
          pltpu.VMEM_SHARED((NPAD, 128), jnp.float32),
          pltpu.SemaphoreType.DMA,
          pltpu.SemaphoreType.DMA,
          pltpu.SemaphoreType.DMA,
          pltpu.SemaphoreType.DMA,
      ],
  )
  def k(x_hbm, e_hbm, src_hbm, dst_hbm, out_hbm,
        src_v, dst_v, x_a, e_a, x_b, e_b, agg_s, ls_a, ls_b, ss_a, ss_b):
    cid = lax.axis_index("c")
    sid = lax.axis_index("s")

    def zrow(r, _):
      for kk in range(KF):
        x_a[r, pl.ds(kk * 16, 16)] = jnp.zeros((16,), jnp.float32)
      return 0
    lax.fori_loop(0, C, zrow, 0)
    zbase = sid * RPT
    for t in range(8):
      pltpu.sync_copy(x_a, agg_s.at[pl.ds(zbase + t * C, C)])
    plsc.subcore_barrier()

    ebase0 = cid * cstride + sid * sstride

    def superstep(b, _):
      flat = (cid * NS + sid) * NSB + b
      pltpu.sync_copy(src_hbm.at[flat], src_v)
      pltpu.sync_copy(dst_hbm.at[flat], dst_v)
      eb = ebase0 + b * SB * C

      def issue_loads(i, xbuf, ebuf, lsem):
        pltpu.async_copy(x_hbm.at[src_v.at[i]], xbuf, lsem)
        pltpu.async_copy(e_hbm.at[pl.ds(eb + i * C, C)], ebuf, lsem)

      def wait_loads(xbuf, ebuf, lsem):
        pltpu.make_async_copy(x_hbm.at[src_v.at[0]], xbuf, lsem).wait()
        pltpu.make_async_copy(e_hbm.at[pl.ds(0, C)], ebuf, lsem).wait()

      def compute(xbuf, ebuf):
        def crow(r4, _):
          r = r4 * 4
          for dr in range(4):
            for kk in range(KF):
              sl = pl.ds(kk * 16, 16)
              ebuf[r + dr, sl] = jnp.maximum(
                  xbuf[r + dr, sl] + ebuf[r + dr, sl], 0.0)
          return 0
        lax.fori_loop(0, C // 4, crow, 0)

      def issue_scatter(i, ebuf, ssem):
        pltpu.async_copy(ebuf, agg_s.at[dst_v.at[i]], ssem, add=True)

      def wait_scatter(ebuf, ssem):
        pltpu.make_async_copy(ebuf, agg_s.at[dst_v.at[0]], ssem).wait()

      issue_loads(0, x_a, e_a, ls_a)
      wait_loads(x_a, e_a, ls_a)
      issue_loads(1, x_b, e_b, ls_b)
      compute(x_a, e_a)
      issue_scatter(0, e_a, ss_a)
      wait_loads(x_b, e_b, ls_b)
      issue_loads(2, x_a, e_a, ls_a)
      compute(x_b, e_b)
      issue_scatter(1, e_b, ss_b)

      def pair(kk, _):
        j0 = 2 * kk
        wait_loads(x_a, e_a, ls_a)
        issue_loads(j0 + 1, x_b, e_b, ls_b)
        wait_scatter(e_a, ss_a)
        compute(x_a, e_a)
        issue_scatter(j0, e_a, ss_a)
        wait_loads(x_b, e_b, ls_b)
        issue_loads(j0 + 2, x_a, e_a, ls_a)
        wait_scatter(e_b, ss_b)
        compute(x_b, e_b)
        issue_scatter(j0 + 1, e_b, ss_b)
        return 0
      lax.fori_loop(1, 12, pair, 0)

      wait_loads(x_a, e_a, ls_a)
      wait_scatter(e_a, ss_a)
      compute(x_a, e_a)
      issue_scatter(24, e_a, ss_a)
      wait_scatter(e_b, ss_b)
      wait_scatter(e_a, ss_a)
      return 0
    lax.fori_loop(0, NSB, superstep, 0)
    plsc.subcore_barrier()

    wbase = cid * NPAD + sid * RPT
    for t in range(8):
      pltpu.sync_copy(agg_s.at[pl.ds(zbase + t * C, C)], x_a)
      pltpu.sync_copy(x_a, out_hbm.at[pl.ds(wbase + t * C, C)])

  return k(xsrc, eee, srcidx, dstidx)


def _mlp1(x, alo, ahi, W11, b11, W12, b12, sc1, sh1):
  def body(x_ref, alo_ref, ahi_ref, w1_ref, b1_ref, w2_ref, b2_ref,
           sc_ref, sh_ref, o_ref):
    h = x_ref[...] + alo_ref[...] + ahi_ref[...]
    t = jax.nn.relu(jnp.dot(h, w1_ref[...], preferred_element_type=jnp.float32) + b1_ref[...])
    u = jnp.dot(t, w2_ref[...], preferred_element_type=jnp.float32) + b2_ref[...]
    o_ref[...] = sc_ref[...] * jax.nn.relu(u) + sh_ref[...]

  return pl.pallas_call(
      body,
      grid=(NNB, NC),
      in_specs=[
          pl.BlockSpec((BN_, 128), lambda j, c: (j, 0)),
          pl.BlockSpec((BN_, 128), lambda j, c: (j, 0)),
          pl.BlockSpec((BN_, 128), lambda j, c: (j, 0)),
          pl.BlockSpec((128, 256), lambda j, c: (0, 0)),
          pl.BlockSpec((1, 256), lambda j, c: (0, 0)),
          pl.BlockSpec((256, 128), lambda j, c: (0, c)),
          pl.BlockSpec((1, 128), lambda j, c: (0, c)),
          pl.BlockSpec((1, 128), lambda j, c: (0, c)),
          pl.BlockSpec((1, 128), lambda j, c: (0, c)),
      ],
      out_specs=pl.BlockSpec((BN_, 128), lambda j, c: (c * NNB + j, 0)),
      out_shape=jax.ShapeDtypeStruct((2 * N, 128), jnp.float32),
  )(x, alo, ahi, W11, b11.reshape(1, 256), W12, b12.reshape(1, 256),
    sc1.reshape(1, 256), sh1.reshape(1, 256))


def _mlp2_final(h1, alo, ahi, W21, b21, W22, b22, sc2, sh2, Wl, bl):
  def body(hlo_ref, hhi_ref, alo_ref, ahi_ref, w1_ref, b1_ref, w2_ref, b2_ref,
           sc_ref, sh_ref, wl_ref, bl_ref, o_ref):
    h = jnp.concatenate([hlo_ref[...], hhi_ref[...]], axis=1)
    a = jnp.concatenate([alo_ref[...], ahi_ref[...]], axis=1)
    z = h + a
    t = jax.nn.relu(jnp.dot(z, w1_ref[...], preferred_element_type=jnp.float32) + b1_ref[...])
    u = jnp.dot(t, w2_ref[...], preferred_element_type=jnp.float32) + b2_ref[...]
    v = sc_ref[...] * jax.nn.relu(u) + sh_ref[...]
    o_ref[...] = jnp.dot(v, wl_ref[...], preferred_element_type=jnp.float32) + bl_ref[...]

  return pl.pallas_call(
      body,
      grid=(NNB,),
      in_specs=[
          pl.BlockSpec((BN_, 128), lambda j: (j, 0)),
          pl.BlockSpec((BN_, 128), lambda j: (NNB + j, 0)),
          pl.BlockSpec((BN_, 128), lambda j: (j, 0)),
          pl.BlockSpec((BN_, 128), lambda j: (j, 0)),
          pl.BlockSpec((256, 256), lambda j: (0, 0)),
          pl.BlockSpec((1, 256), lambda j: (0, 0)),
          pl.BlockSpec((256, 256), lambda j: (0, 0)),
          pl.BlockSpec((1, 256), lambda j: (0, 0)),
          pl.BlockSpec((1, 256), lambda j: (0, 0)),
          pl.BlockSpec((1, 256), lambda j: (0, 0)),
          pl.BlockSpec((256, 128), lambda j: (0, 0)),
          pl.BlockSpec((1, 128), lambda j: (0, 0)),
      ],
      out_specs=pl.BlockSpec((BN_, 128), lambda j: (j, 0)),
      out_shape=jax.ShapeDtypeStruct((N, 128), jnp.float32),
  )(h1, h1, alo, ahi, W21, b21.reshape(1, 256), W22, b22.reshape(1, 256),
    sc2.reshape(1, 256), sh2.reshape(1, 256), Wl, bl.reshape(1, 128))


def kernel(x, edge_index, edge_attr, We1, be1, W11, b11, W12, b12,
           We2, be2, W21, b21, W22, b22, g1, bb1, g2, bb2, Wl, bl):
  src = edge_index[0]
  dst = edge_index[1]

  nch1 = E // (NC * NS) // C
  src1 = src.reshape(-1, 25, C)
  dst1 = dst.reshape(-1, 25, C)

  src_r = src.reshape(NS, NCH, C)
  src2 = jnp.stack([src_r, src_r + N]).reshape(-1, 25, C)
  dst_r = dst.reshape(NS, NCH, C)
  dst2 = jnp.stack([dst_r, dst_r]).reshape(-1, 25, C)

  inv = 1.0 / jnp.sqrt(jnp.float32(1.0 + 1e-5))
  sc1, sh1 = g1 * inv, bb1
  sc2, sh2 = g2 * inv, bb2

  e1 = _edge1(edge_attr, We1, be1)
  e2 = _edge2(edge_attr, We2, be2)

  agg1 = _sc_aggregate(x, e1, src1, dst1, nch1, E // NC, E // (NC * NS))
  h1 = _mlp1(x, agg1[:N], agg1[NPAD:NPAD + N],
             W11, b11, W12, b12, sc1, sh1)

  agg2 = _sc_aggregate(h1, e2, src2, dst2, NCH, E, EPS)
  return _mlp2_final(h1, agg2[:N], agg2[NPAD:NPAD + N],
                     W21, b21, W22, b22, sc2, sh2, Wl, bl)
